# baseline (speedup 1.0000x reference)
import functools

import jax
import jax.numpy as jnp
from jax import lax
from jax.experimental import pallas as pl
from jax.experimental.pallas import tpu as pltpu
from jax.experimental.pallas import tpu_sc as plsc

_L = 14
_P = _L * _L
_R = 5
_THRESHOLD = 0.6
_CIO = 256
_NCH = 4


def _combine(a, b):
    av, ai = a
    bv, bi = b
    repl = (bv > av) | ((bv == av) & (bi < ai))
    return jnp.where(repl, bv, av), jnp.where(repl, bi, ai)


def _nms_body(bpw, h_hbm, out1_hbm, out2_hbm, land1, land2, heat_v, sem_out):
    info = plsc.get_sparse_core_info()
    nc, lanes_n = info.num_cores, info.num_lanes
    ngroups = bpw // lanes_n

    wid = lax.axis_index("s") * nc + lax.axis_index("c")
    pltpu.sync_copy(h_hbm.at[wid], land1)

    lanes = lax.iota(jnp.int32, lanes_n)
    zeros_i = jnp.zeros((lanes_n,), jnp.int32)
    zeros_f = jnp.zeros((lanes_n,), jnp.float32)
    ones_f = jnp.full((lanes_n,), 1.0, jnp.float32)

    nchunk = (_P + lanes_n - 1) // lanes_n

    def repack(e, c):
        dst = e * _P
        for k in range(nchunk):
            v = land1[e, pl.ds(k * lanes_n, lanes_n)]
            plsc.store_scatter(heat_v, [dst + k * lanes_n + lanes], v)
        return c

    lax.fori_loop(0, bpw, repack, 0)

    u0 = 13 * lanes
    iv0 = u0 // _L
    jv0 = u0 - iv0 * _L

    def group_body(g, carry):
        exv = g * lanes_n + lanes
        bvec = exv * _P

        ims = []
        ci_glob = bvec
        for r in range(4):

            accs = [
                (jnp.full((lanes_n,), _THRESHOLD, jnp.float32), bvec)
                for _ in range(_NCH)
            ]
            for p in range(_P):
                q = p % _NCH
                cm, ci = accs[q]
                idx = bvec + p
                v = plsc.load_gather(heat_v, [idx])
                cond = v > cm
                accs[q] = (
                    jnp.where(cond, v, cm),
                    jnp.where(cond, idx, ci),
                )
            while len(accs) > 1:
                nxt = [
                    _combine(accs[2 * t], accs[2 * t + 1])
                    for t in range(len(accs) // 2)
                ]
                if len(accs) % 2:
                    nxt.append(accs[-1])
                accs = nxt
            _, ci_glob = accs[0]
            im = ci_glob - bvec
            ims.append(im)

            if r < 3:
                x = im // _L
                y = im - x * _L
                okys = []
                for dyj in range(2 * _R):
                    yn = y + (dyj - _R)
                    okys.append((yn >= 0) & (yn < _L))

                def sup_body(t, ci):
                    dx = t - _R
                    xn = x + dx
                    okx = (xn >= 0) & (xn < _L)
                    row_t = ci + dx * _L
                    for dyj in range(2 * _R):
                        ok = okx & okys[dyj]
                        tgt = row_t + (dyj - _R)
                        plsc.store_scatter(heat_v, [tgt], zeros_f, mask=ok)
                    return ci

                lax.fori_loop(0, 2 * _R, sup_body, ci_glob)

        xs = [im // _L for im in ims]
        ys = [im - (im // _L) * _L for im in ims]
        pairs = [(0, 1), (0, 2), (0, 3), (1, 2), (1, 3), (2, 3)]
        best = jnp.full((lanes_n,), -1, jnp.int32)
        c1x, c1y, c2x, c2y = xs[0], ys[0], xs[1], ys[1]
        for a, b in pairs:
            dxx = xs[b] - xs[a]
            dyy = ys[b] - ys[a]
            d = dxx * dxx + dyy * dyy
            cond = d > best
            best = jnp.where(cond, d, best)
            c1x = jnp.where(cond, xs[a], c1x)
            c1y = jnp.where(cond, ys[a], c1y)
            c2x = jnp.where(cond, xs[b], c2x)
            c2y = jnp.where(cond, ys[b], c2y)

        ax = 2 * (c2x - c1x)
        ay = 2 * (c2y - c1y)
        kk = c2x * c2x + c2y * c2y - c1x * c1x - c1y * c1y

        def vor_block(s, st):
            iv, jv = st
            for _ in range(_L):
                lhs = iv * ax + jv * ay - kk
                m = lhs < 0
                m1 = jnp.where(m, ones_f, zeros_f)
                m2 = ones_f - m1
                pv = iv * _L + jv
                plsc.store_scatter(land1, [exv, pv], m1)
                plsc.store_scatter(land2, [exv, pv], m2)
                jz = jv == 0
                jv = jnp.where(jz, _L - 1, jv - 1)
                iv = jnp.where(jz, iv, iv + 1)
                iv = jnp.where(iv >= _L, iv - _L, iv)
            return iv, jv

        lax.fori_loop(0, _L, vor_block, (iv0, jv0))

        pltpu.make_async_copy(
            land1.at[pl.ds(g * lanes_n, lanes_n)],
            out1_hbm.at[wid, pl.ds(g * lanes_n, lanes_n)],
            sem_out,
        ).start()
        pltpu.make_async_copy(
            land2.at[pl.ds(g * lanes_n, lanes_n)],
            out2_hbm.at[wid, pl.ds(g * lanes_n, lanes_n)],
            sem_out,
        ).start()
        return carry

    lax.fori_loop(0, ngroups, group_body, 0)

    def drain_out(g, c):
        pltpu.make_async_copy(
            land1.at[pl.ds(0, lanes_n)],
            out1_hbm.at[0, pl.ds(0, lanes_n)],
            sem_out,
        ).wait()
        pltpu.make_async_copy(
            land2.at[pl.ds(0, lanes_n)],
            out2_hbm.at[0, pl.ds(0, lanes_n)],
            sem_out,
        ).wait()
        return c

    lax.fori_loop(0, ngroups, drain_out, 0)


@functools.partial(jax.jit, static_argnums=(1,))
def _nms_run(h3, bpw):
    nw = h3.shape[0]
    mesh = plsc.VectorSubcoreMesh(core_axis_name="c", subcore_axis_name="s")
    out = pl.kernel(
        functools.partial(_nms_body, bpw),
        out_type=(
            jax.ShapeDtypeStruct((nw, bpw, _CIO), jnp.float32),
            jax.ShapeDtypeStruct((nw, bpw, _CIO), jnp.float32),
        ),
        mesh=mesh,
        compiler_params=pltpu.CompilerParams(needs_layout_passes=False),
        scratch_types=[
            pltpu.VMEM((bpw, _CIO), jnp.float32),
            pltpu.VMEM((bpw, _CIO), jnp.float32),
            pltpu.VMEM((bpw * _P,), jnp.float32),
            pltpu.SemaphoreType.DMA,
        ],
    )(h3)
    return out


def kernel(heatmap):
    b = heatmap.shape[0]
    info = plsc.get_sparse_core_info()
    nw = info.num_cores * info.num_subcores
    bpw = b // nw
    h2 = jnp.pad(heatmap.reshape(b, _P), ((0, 0), (0, _CIO - _P)))
    o1, o2 = _nms_run(h2.reshape(nw, bpw, _CIO), bpw)
    o1 = o1.reshape(b, _CIO)[:, :_P].reshape(b, 1, _L, _L)
    o2 = o2.reshape(b, _CIO)[:, :_P].reshape(b, 1, _L, _L)
    return (o1, o2)


# speedup vs baseline: 1.1764x; 1.1764x over previous
import functools

import jax
import jax.numpy as jnp
from jax import lax
from jax.experimental import pallas as pl
from jax.experimental.pallas import tpu as pltpu
from jax.experimental.pallas import tpu_sc as plsc

_L = 14
_P = _L * _L
_R = 5
_THRESHOLD = 0.6
_CIO = 256
_NCH = 4


def _combine(a, b):
    av, ai = a
    bv, bi = b
    repl = (bv > av) | ((bv == av) & (bi < ai))
    return jnp.where(repl, bv, av), jnp.where(repl, bi, ai)


def _nms_body(bpw, h_hbm, out1_hbm, out2_hbm, land1, land2, heat_v, sem_out):
    info = plsc.get_sparse_core_info()
    nc, lanes_n = info.num_cores, info.num_lanes
    ngroups = bpw // lanes_n

    wid = lax.axis_index("s") * nc + lax.axis_index("c")
    pltpu.sync_copy(h_hbm.at[wid], land1)

    lanes = lax.iota(jnp.int32, lanes_n)
    zeros_i = jnp.zeros((lanes_n,), jnp.int32)
    zeros_f = jnp.zeros((lanes_n,), jnp.float32)
    ones_f = jnp.full((lanes_n,), 1.0, jnp.float32)

    nchunk = (_P + lanes_n - 1) // lanes_n

    def repack(e, c):
        dst = e * _P
        for k in range(nchunk):
            v = land1[e, pl.ds(k * lanes_n, lanes_n)]
            plsc.store_scatter(heat_v, [dst + k * lanes_n + lanes], v)
        return c

    lax.fori_loop(0, bpw, repack, 0)

    u0 = 13 * lanes
    iv0 = u0 // _L
    jv0 = u0 - iv0 * _L

    def group_body(g, carry):
        exv = g * lanes_n + lanes
        bvec = exv * _P

        ims = []
        ci_glob = bvec
        for r in range(4):

            def scan_rows(i, accs):
                row = bvec + i * (2 * _L)
                out = list(accs)
                for j in range(2 * _L):
                    q = j % _NCH
                    cm, ci = out[q]
                    idx = row + j
                    v = plsc.load_gather(heat_v, [idx])
                    cond = v > cm
                    out[q] = (
                        jnp.where(cond, v, cm),
                        jnp.where(cond, idx, ci),
                    )
                return tuple(out)

            init = tuple(
                (jnp.full((lanes_n,), _THRESHOLD, jnp.float32), bvec)
                for _ in range(_NCH)
            )
            accs = list(lax.fori_loop(0, _L // 2, scan_rows, init))
            while len(accs) > 1:
                nxt = [
                    _combine(accs[2 * t], accs[2 * t + 1])
                    for t in range(len(accs) // 2)
                ]
                if len(accs) % 2:
                    nxt.append(accs[-1])
                accs = nxt
            _, ci_glob = accs[0]
            im = ci_glob - bvec
            ims.append(im)

            if r < 3:
                x = im // _L
                y = im - x * _L
                okys = []
                for dyj in range(2 * _R):
                    yn = y + (dyj - _R)
                    okys.append((yn >= 0) & (yn < _L))

                def sup_body(t, ci):
                    dx = t - _R
                    xn = x + dx
                    okx = (xn >= 0) & (xn < _L)
                    row_t = ci + dx * _L
                    for dyj in range(2 * _R):
                        ok = okx & okys[dyj]
                        tgt = row_t + (dyj - _R)
                        plsc.store_scatter(heat_v, [tgt], zeros_f, mask=ok)
                    return ci

                lax.fori_loop(0, 2 * _R, sup_body, ci_glob)

        xs = [im // _L for im in ims]
        ys = [im - (im // _L) * _L for im in ims]
        pairs = [(0, 1), (0, 2), (0, 3), (1, 2), (1, 3), (2, 3)]
        best = jnp.full((lanes_n,), -1, jnp.int32)
        c1x, c1y, c2x, c2y = xs[0], ys[0], xs[1], ys[1]
        for a, b in pairs:
            dxx = xs[b] - xs[a]
            dyy = ys[b] - ys[a]
            d = dxx * dxx + dyy * dyy
            cond = d > best
            best = jnp.where(cond, d, best)
            c1x = jnp.where(cond, xs[a], c1x)
            c1y = jnp.where(cond, ys[a], c1y)
            c2x = jnp.where(cond, xs[b], c2x)
            c2y = jnp.where(cond, ys[b], c2y)

        ax = 2 * (c2x - c1x)
        ay = 2 * (c2y - c1y)
        kk = c2x * c2x + c2y * c2y - c1x * c1x - c1y * c1y

        def vor_block(s, st):
            iv, jv = st
            for _ in range(_L):
                lhs = iv * ax + jv * ay - kk
                m = lhs < 0
                m1 = jnp.where(m, ones_f, zeros_f)
                m2 = ones_f - m1
                pv = iv * _L + jv
                plsc.store_scatter(land1, [exv, pv], m1)
                plsc.store_scatter(land2, [exv, pv], m2)
                jz = jv == 0
                jv = jnp.where(jz, _L - 1, jv - 1)
                iv = jnp.where(jz, iv, iv + 1)
                iv = jnp.where(iv >= _L, iv - _L, iv)
            return iv, jv

        lax.fori_loop(0, _L, vor_block, (iv0, jv0))

        pltpu.make_async_copy(
            land1.at[pl.ds(g * lanes_n, lanes_n)],
            out1_hbm.at[wid, pl.ds(g * lanes_n, lanes_n)],
            sem_out,
        ).start()
        pltpu.make_async_copy(
            land2.at[pl.ds(g * lanes_n, lanes_n)],
            out2_hbm.at[wid, pl.ds(g * lanes_n, lanes_n)],
            sem_out,
        ).start()
        return carry

    lax.fori_loop(0, ngroups, group_body, 0)

    def drain_out(g, c):
        pltpu.make_async_copy(
            land1.at[pl.ds(0, lanes_n)],
            out1_hbm.at[0, pl.ds(0, lanes_n)],
            sem_out,
        ).wait()
        pltpu.make_async_copy(
            land2.at[pl.ds(0, lanes_n)],
            out2_hbm.at[0, pl.ds(0, lanes_n)],
            sem_out,
        ).wait()
        return c

    lax.fori_loop(0, ngroups, drain_out, 0)


@functools.partial(jax.jit, static_argnums=(1,))
def _nms_run(h3, bpw):
    nw = h3.shape[0]
    mesh = plsc.VectorSubcoreMesh(core_axis_name="c", subcore_axis_name="s")
    out = pl.kernel(
        functools.partial(_nms_body, bpw),
        out_type=(
            jax.ShapeDtypeStruct((nw, bpw, _CIO), jnp.float32),
            jax.ShapeDtypeStruct((nw, bpw, _CIO), jnp.float32),
        ),
        mesh=mesh,
        compiler_params=pltpu.CompilerParams(needs_layout_passes=False),
        scratch_types=[
            pltpu.VMEM((bpw, _CIO), jnp.float32),
            pltpu.VMEM((bpw, _CIO), jnp.float32),
            pltpu.VMEM((bpw * _P,), jnp.float32),
            pltpu.SemaphoreType.DMA,
        ],
    )(h3)
    return out


def kernel(heatmap):
    b = heatmap.shape[0]
    info = plsc.get_sparse_core_info()
    nw = info.num_cores * info.num_subcores
    bpw = b // nw
    h2 = jnp.pad(heatmap.reshape(b, _P), ((0, 0), (0, _CIO - _P)))
    o1, o2 = _nms_run(h2.reshape(nw, bpw, _CIO), bpw)
    o1 = o1.reshape(b, _CIO)[:, :_P].reshape(b, 1, _L, _L)
    o2 = o2.reshape(b, _CIO)[:, :_P].reshape(b, 1, _L, _L)
    return (o1, o2)
